# Initial kernel scaffold; baseline (speedup 1.0000x reference)
import jax
import jax.numpy as jnp
from jax.experimental import pallas as pl


def kernel(x, weight, lora_A, lora_B):
    raise NotImplementedError("write your pallas kernel here")


# speedup vs baseline: 3.2167x; 3.2167x over previous
import functools

import jax
import jax.numpy as jnp
from jax import lax
from jax.experimental import pallas as pl
from jax.experimental.pallas import tpu as pltpu
from jax.experimental.pallas import tpu_sc as plsc

_DIM = 64
_RANK = 128
_SCALE = 2.0

_CHUNK = 128


def _gather_body(tpw, nc, idxp_hbm, idx_hbm, w2_hbm, a_hbm, wout_hbm,
                 aout_hbm, idxp_v, idx_v, wbuf, abuf, sem):
    wid = lax.axis_index("s") * nc + lax.axis_index("c")
    base = wid * tpw

    def body(i, carry):
        off = base + i * _CHUNK
        pltpu.sync_copy(idxp_hbm.at[pl.ds(off, _CHUNK)], idxp_v)
        pltpu.sync_copy(idx_hbm.at[pl.ds(off, _CHUNK)], idx_v)
        cw = pltpu.async_copy(w2_hbm.at[idxp_v], wbuf, sem)
        ca = pltpu.async_copy(a_hbm.at[idx_v], abuf, sem)
        cw.wait()
        ca.wait()
        pltpu.sync_copy(wbuf, wout_hbm.at[pl.ds(off, _CHUNK)])
        pltpu.sync_copy(abuf, aout_hbm.at[pl.ds(off, _CHUNK)])
        return carry

    lax.fori_loop(0, tpw // _CHUNK, body, 0)


def _lora_mm_body(i_ref, eye_ref, w_ref, a_ref, b_ref, o_ref):
    p_row = lax.rem(i_ref[0], 2).astype(jnp.float32)
    p_col = lax.dot_general(eye_ref[...], p_row,
                            (((1,), (1,)), ((), ())),
                            preferred_element_type=jnp.float32)
    lo = w_ref[:, :_DIM]
    hi = w_ref[:, _DIM:]
    wsel = lo + p_col * (hi - lo)
    o_ref[...] = wsel + _SCALE * jnp.dot(
        a_ref[...], b_ref[...], preferred_element_type=jnp.float32)


def kernel(x, weight, lora_A, lora_B):
    b, s = x.shape
    n = b * s
    idx = x.reshape(n)
    idx_pair = idx // 2
    vocab = weight.shape[0]
    w2 = weight.reshape(vocab // 2, 2 * _DIM)

    info = plsc.get_sparse_core_info()
    nc, ns = info.num_cores, info.num_subcores
    nw = nc * ns
    tpw = n // nw

    gather = pl.kernel(
        functools.partial(_gather_body, tpw, nc),
        mesh=plsc.VectorSubcoreMesh(core_axis_name="c", subcore_axis_name="s"),
        out_type=[
            jax.ShapeDtypeStruct((n, 2 * _DIM), jnp.float32),
            jax.ShapeDtypeStruct((n, _RANK), jnp.float32),
        ],
        scratch_types=[
            pltpu.VMEM((_CHUNK,), jnp.int32),
            pltpu.VMEM((_CHUNK,), jnp.int32),
            pltpu.VMEM((_CHUNK, 2 * _DIM), jnp.float32),
            pltpu.VMEM((_CHUNK, _RANK), jnp.float32),
            pltpu.SemaphoreType.DMA,
        ],
    )
    w_rows, a_rows = gather(idx_pair, idx, w2, lora_A)

    t = 512
    eye = jnp.eye(t, dtype=jnp.float32)
    out = pl.pallas_call(
        _lora_mm_body,
        grid=(n // t,),
        in_specs=[
            pl.BlockSpec((1, 1, t), lambda i: (i, 0, 0)),
            pl.BlockSpec((t, t), lambda i: (0, 0)),
            pl.BlockSpec((t, 2 * _DIM), lambda i: (i, 0)),
            pl.BlockSpec((t, _RANK), lambda i: (i, 0)),
            pl.BlockSpec((_RANK, _DIM), lambda i: (0, 0)),
        ],
        out_specs=pl.BlockSpec((t, _DIM), lambda i: (i, 0)),
        out_shape=jax.ShapeDtypeStruct((n, _DIM), jnp.float32),
    )(idx.reshape(n // t, 1, t), eye, w_rows, a_rows, lora_B)

    return out.reshape(b, s, _DIM)
